# baseline (speedup 1.0000x reference)
import functools

import jax
import jax.numpy as jnp
from jax import lax
from jax.experimental import pallas as pl
from jax.experimental.pallas import tpu as pltpu
from jax.experimental.pallas import tpu_sc as plsc

BATCH = 16384
EMBED = 32
LANES = 16


@functools.lru_cache(maxsize=None)
def _make_kernel(num_cores: int, num_subcores: int):
    num_workers = num_cores * num_subcores
    b_per_w = BATCH // num_workers
    mesh = plsc.VectorSubcoreMesh(core_axis_name="c", subcore_axis_name="s")

    @functools.partial(
        pl.kernel,
        out_type=jax.ShapeDtypeStruct((BATCH,), jnp.float32),
        mesh=mesh,
        compiler_params=pltpu.CompilerParams(needs_layout_passes=False,
                                             use_tc_tiling_on_sc=False),
        scratch_types=[
            pltpu.VMEM((b_per_w,), jnp.int32),
            pltpu.VMEM((b_per_w,), jnp.int32),
            pltpu.VMEM((b_per_w, EMBED), jnp.float32),
            pltpu.VMEM((b_per_w, EMBED), jnp.float32),
            pltpu.VMEM((b_per_w,), jnp.float32),
            pltpu.SemaphoreType.DMA,
        ],
    )
    def sc_kernel(uids_hbm, iids_hbm, utab_hbm, itab_hbm, out_hbm,
                  uidx_v, iidx_v, urows_v, irows_v, out_v, sem):
        wid = lax.axis_index("s") * num_cores + lax.axis_index("c")
        base = wid * b_per_w
        pltpu.sync_copy(uids_hbm.at[pl.ds(base, b_per_w)], uidx_v)
        pltpu.sync_copy(iids_hbm.at[pl.ds(base, b_per_w)], iidx_v)
        cu = pltpu.async_copy(utab_hbm.at[uidx_v], urows_v, sem)
        ci = pltpu.async_copy(itab_hbm.at[iidx_v], irows_v, sem)
        cu.wait()
        ci.wait()

        lanes = lax.iota(jnp.int32, LANES)

        def body(g, carry):
            acc = jnp.zeros((LANES,), jnp.float32)
            for j in range(LANES):
                r = g * LANES + j
                s = (urows_v[r, pl.ds(0, LANES)] * irows_v[r, pl.ds(0, LANES)]
                     + urows_v[r, pl.ds(LANES, LANES)]
                     * irows_v[r, pl.ds(LANES, LANES)])
                acc = jnp.where(lanes == j, jnp.sum(s), acc)
            out_v[pl.ds(g * LANES, LANES)] = acc
            return carry

        lax.fori_loop(0, b_per_w // LANES, body, 0)
        pltpu.sync_copy(out_v, out_hbm.at[pl.ds(base, b_per_w)])

    return sc_kernel


def kernel(user_ids, item_ids, user_table, item_table):
    info = plsc.get_sparse_core_info()
    sc_kernel = _make_kernel(info.num_cores, info.num_subcores)
    return sc_kernel(user_ids.astype(jnp.int32), item_ids.astype(jnp.int32),
                     user_table, item_table)
